# Initial kernel scaffold; baseline (speedup 1.0000x reference)
import jax
import jax.numpy as jnp
from jax.experimental import pallas as pl


def kernel(sequence, emb_table, pos_encoding, training, mask):
    raise NotImplementedError("write your pallas kernel here")


# speedup vs baseline: 4.3659x; 4.3659x over previous
import functools

import jax
import jax.numpy as jnp
from jax import lax
from jax.experimental import pallas as pl
from jax.experimental.pallas import tpu as pltpu
from jax.experimental.pallas import tpu_sc as plsc

VOCAB = 100000
D = 128
B = 1024
L = 200
LANES = 16
NC = 2
NS = 16
NW = NC * NS
SEQ_PER_W = B // NW
SCALE = float(D) ** 0.5
LA, LB = 104, 96


def _body(seq_hbm, table_hbm, pos_hbm, out_hbm,
          idx_all, pos_v, rows_v, sem_a, sem_b):
    wid = lax.axis_index("s") * NC + lax.axis_index("c")
    pltpu.sync_copy(pos_hbm, pos_v)
    pltpu.sync_copy(seq_hbm.at[pl.ds(wid * SEQ_PER_W * L, SEQ_PER_W * L)],
                    idx_all)

    def one_seq(j, carry):
        s = wid * SEQ_PER_W + j
        cp_a = pltpu.async_copy(table_hbm.at[idx_all.at[pl.ds(j * L, LA)]],
                                rows_v.at[pl.ds(0, LA)], sem_a)
        cp_b = pltpu.async_copy(table_hbm.at[idx_all.at[pl.ds(j * L + LA, LB)]],
                                rows_v.at[pl.ds(LA, LB)], sem_b)
        cp_a.wait()
        cp_b.wait()

        def one_row(r, carry2):
            for c in range(D // LANES):
                sl = pl.ds(c * LANES, LANES)
                rows_v[r, sl] = rows_v[r, sl] * SCALE + pos_v[r, sl]
            return carry2

        lax.fori_loop(0, L, one_row, 0, unroll=False)
        pltpu.sync_copy(rows_v, out_hbm.at[s])
        return carry

    lax.fori_loop(0, SEQ_PER_W, one_seq, 0, unroll=False)


@jax.jit
def _pre_process(sequence, emb_table, pos_slice):
    f = functools.partial(
        pl.kernel,
        out_type=jax.ShapeDtypeStruct((B, L, D), jnp.float32),
        mesh=plsc.VectorSubcoreMesh(core_axis_name="c", subcore_axis_name="s"),
        scratch_types=[
            pltpu.VMEM((SEQ_PER_W * L,), jnp.int32),
            pltpu.VMEM((L, D), jnp.float32),
            pltpu.VMEM((L, D), jnp.float32),
            pltpu.SemaphoreType.DMA,
            pltpu.SemaphoreType.DMA,
        ],
    )(_body)
    return f(sequence, emb_table, pos_slice)


def kernel(sequence, emb_table, pos_encoding, training=False, mask=None):
    seq = sequence.astype(jnp.int32).reshape(B * L)
    pos_slice = pos_encoding[0, :L, :].astype(jnp.float32)
    return _pre_process(seq, emb_table, pos_slice)


# speedup vs baseline: 7.3027x; 1.6727x over previous
import functools

import jax
import jax.numpy as jnp
from jax import lax
from jax.experimental import pallas as pl
from jax.experimental.pallas import tpu as pltpu
from jax.experimental.pallas import tpu_sc as plsc

VOCAB = 100000
D = 128
B = 1024
L = 200
LANES = 16
NC = 2
NS = 16
NW = NC * NS
SEQ_PER_W = B // NW
SCALE = float(D) ** 0.5
LA, LB = 104, 96
NBUF = 3


def _body(seq_hbm, table_hbm, pos_hbm, out_hbm,
          idx_all, pos_v, rows_v, sg0, sg1, sg2, sw0, sw1, sw2):
    sem_g = [sg0, sg1, sg2]
    sem_w = [sw0, sw1, sw2]
    wid = lax.axis_index("s") * NC + lax.axis_index("c")
    base = wid * SEQ_PER_W
    pltpu.sync_copy(pos_hbm, pos_v)
    pltpu.sync_copy(seq_hbm.at[pl.ds(base * L, SEQ_PER_W * L)], idx_all)

    def gather_pair(j, k):
        pltpu.async_copy(table_hbm.at[idx_all.at[pl.ds(j * L, LA)]],
                         rows_v.at[k, pl.ds(0, LA)], sem_g[k])
        pltpu.async_copy(table_hbm.at[idx_all.at[pl.ds(j * L + LA, LB)]],
                         rows_v.at[k, pl.ds(LA, LB)], sem_g[k])

    def gather_wait(j, k):
        pltpu.make_async_copy(table_hbm.at[idx_all.at[pl.ds(j * L, LA)]],
                              rows_v.at[k, pl.ds(0, LA)], sem_g[k]).wait()
        pltpu.make_async_copy(table_hbm.at[idx_all.at[pl.ds(j * L + LA, LB)]],
                              rows_v.at[k, pl.ds(LA, LB)], sem_g[k]).wait()

    def write_start(j, k):
        pltpu.async_copy(rows_v.at[k], out_hbm.at[base + j], sem_w[k])

    def write_wait(j, k):
        pltpu.make_async_copy(rows_v.at[k], out_hbm.at[base + j],
                              sem_w[k]).wait()

    def compute(k):
        def one_row(r, carry):
            for c in range(D // LANES):
                sl = pl.ds(c * LANES, LANES)
                rows_v[k, r, sl] = rows_v[k, r, sl] * SCALE + pos_v[r, sl]
            return carry

        lax.fori_loop(0, L, one_row, 0, unroll=False)

    def step(j, t, drain_write, prefetch):
        gather_wait(j, t)
        compute(t)
        write_start(j, t)
        kn = (t + 2) % NBUF
        if drain_write:
            write_wait(j - 1, kn)
        if prefetch:
            gather_pair(j + 2, kn)

    gather_pair(0, 0)
    gather_pair(1, 1)
    step(0, 0, drain_write=False, prefetch=True)
    step(1, 1, drain_write=True, prefetch=True)
    step(2, 2, drain_write=True, prefetch=True)

    def group(jj, carry):
        for t in range(NBUF):
            step(jj * NBUF + t, t, drain_write=True, prefetch=True)
        return carry

    lax.fori_loop(1, SEQ_PER_W // NBUF, group, 0, unroll=False)
    step(SEQ_PER_W - 2, 0, drain_write=True, prefetch=False)
    step(SEQ_PER_W - 1, 1, drain_write=True, prefetch=False)
    write_wait(SEQ_PER_W - 1, 1)


@jax.jit
def _pre_process(sequence, emb_table, pos_slice):
    f = functools.partial(
        pl.kernel,
        out_type=jax.ShapeDtypeStruct((B, L, D), jnp.float32),
        mesh=plsc.VectorSubcoreMesh(core_axis_name="c", subcore_axis_name="s"),
        scratch_types=[
            pltpu.VMEM((SEQ_PER_W * L,), jnp.int32),
            pltpu.VMEM((L, D), jnp.float32),
            pltpu.VMEM((NBUF, L, D), jnp.float32),
            pltpu.SemaphoreType.DMA,
            pltpu.SemaphoreType.DMA,
            pltpu.SemaphoreType.DMA,
            pltpu.SemaphoreType.DMA,
            pltpu.SemaphoreType.DMA,
            pltpu.SemaphoreType.DMA,
        ],
    )(_body)
    return f(sequence, emb_table, pos_slice)


def kernel(sequence, emb_table, pos_encoding, training=False, mask=None):
    seq = sequence.astype(jnp.int32).reshape(B * L)
    pos_slice = pos_encoding[0, :L, :].astype(jnp.float32)
    return _pre_process(seq, emb_table, pos_slice)
